# Initial kernel scaffold; baseline (speedup 1.0000x reference)
import functools
import jax
import jax.numpy as jnp
from jax import lax
from jax.experimental import pallas as pl
from jax.experimental.pallas import tpu as pltpu
from jax.experimental.pallas import tpu_sc as plsc

B = 16384
V = 11
D = 64
NC, NS = 2, 16
NW = NC * NS
BPW = B // NW
CHUNK = 128
NCH = BPW // CHUNK


def _table_body(e1_ref, e2_ref, out_ref):
    for a in range(V):
        out_ref[pl.ds(a * V, V), pl.ds(0, D)] = jnp.broadcast_to(
            e1_ref[pl.ds(a, 1), :], (V, D))
        out_ref[pl.ds(a * V, V), pl.ds(D, D)] = e2_ref[...]


def _build_table(emb1, emb2):
    return pl.pallas_call(
        _table_body,
        out_shape=jax.ShapeDtypeStruct((128, 2 * D), jnp.float32),
    )(emb1, emb2)


@functools.partial(
    pl.kernel,
    mesh=plsc.VectorSubcoreMesh(core_axis_name="c", subcore_axis_name="s"),
    out_type=jax.ShapeDtypeStruct((NW * NCH, CHUNK, 2 * D), jnp.float32),
    scratch_types=[
        pltpu.VMEM((NCH, CHUNK), jnp.int32),
        pltpu.VMEM((NCH, CHUNK), jnp.int32),
        pltpu.VMEM((NCH, CHUNK), jnp.int32),
        pltpu.VMEM((NCH, CHUNK, 2 * D), jnp.float32),
        pltpu.SemaphoreType.DMA,
    ],
)
def _sc_gather(y_hbm, s_hbm, table_hbm, out_hbm, y_v, s_v, idx_v, rows_v, sem):
    wid = lax.axis_index("s") * NC + lax.axis_index("c")
    base = wid * NCH
    pltpu.sync_copy(y_hbm.at[pl.ds(base, NCH)], y_v)
    pltpu.sync_copy(s_hbm.at[pl.ds(base, NCH)], s_v)
    for c in range(NCH):
        for m in range(CHUNK // 16):
            sl = pl.ds(m * 16, 16)
            idx_v[c, sl] = y_v[c, sl] * V + s_v[c, sl]
    copies = [
        pltpu.async_copy(table_hbm.at[idx_v.at[c]], rows_v.at[c], sem)
        for c in range(NCH)
    ]
    for cp in copies:
        cp.wait()
    pltpu.sync_copy(rows_v, out_hbm.at[pl.ds(base, NCH)])


def kernel(y, s, emb1, emb2):
    table = _build_table(emb1, emb2)
    y2 = y.astype(jnp.int32).reshape(NW * NCH, CHUNK)
    s2 = s.astype(jnp.int32).reshape(NW * NCH, CHUNK)
    out = _sc_gather(y2, s2, table)
    return out.reshape(B, 2 * D)


# speedup vs baseline: 3.7369x; 3.7369x over previous
import functools
import jax
import jax.numpy as jnp
from jax import lax
from jax.experimental import pallas as pl
from jax.experimental.pallas import tpu as pltpu
from jax.experimental.pallas import tpu_sc as plsc

B = 16384
V = 11
D = 64
NC, NS = 2, 16
NW = NC * NS
BPW = B // NW
CHUNK = 128
NCH = BPW // CHUNK


def _table_body(e1_ref, e2_ref, out_ref):
    for a in range(V):
        out_ref[pl.ds(a * V, V), pl.ds(0, D)] = jnp.broadcast_to(
            e1_ref[pl.ds(a, 1), :], (V, D))
        out_ref[pl.ds(a * V, V), pl.ds(D, D)] = e2_ref[...]


def _build_table(emb1, emb2):
    return pl.pallas_call(
        _table_body,
        out_shape=jax.ShapeDtypeStruct((128, 2 * D), jnp.float32),
    )(emb1, emb2)


@functools.cache
def _make_sc_gather():
    @functools.partial(
        pl.kernel,
        mesh=plsc.VectorSubcoreMesh(core_axis_name="c", subcore_axis_name="s"),
        out_type=jax.ShapeDtypeStruct((NW * NCH, CHUNK, 2 * D), jnp.float32),
        scratch_types=[
            pltpu.VMEM((NCH, CHUNK), jnp.int32),
            pltpu.VMEM((NCH, CHUNK), jnp.int32),
            pltpu.VMEM((NCH, CHUNK), jnp.int32),
            pltpu.VMEM((NCH, CHUNK, 2 * D), jnp.float32),
            pltpu.SemaphoreType.DMA,
        ],
    )
    def _sc_gather(y_hbm, s_hbm, table_hbm, out_hbm, y_v, s_v, idx_v, rows_v,
                   sem):
        wid = lax.axis_index("s") * NC + lax.axis_index("c")
        base = wid * NCH
        pltpu.sync_copy(y_hbm.at[pl.ds(base, NCH)], y_v)
        pltpu.sync_copy(s_hbm.at[pl.ds(base, NCH)], s_v)
        for c in range(NCH):
            for m in range(CHUNK // 16):
                sl = pl.ds(m * 16, 16)
                idx_v[c, sl] = y_v[c, sl] * V + s_v[c, sl]
        copies = [
            pltpu.async_copy(table_hbm.at[idx_v.at[c]], rows_v.at[c], sem)
            for c in range(NCH)
        ]
        for cp in copies:
            cp.wait()
        pltpu.sync_copy(rows_v, out_hbm.at[pl.ds(base, NCH)])

    return _sc_gather


def kernel(y, s, emb1, emb2):
    table = _build_table(emb1, emb2)
    y2 = y.astype(jnp.int32).reshape(NW * NCH, CHUNK)
    s2 = s.astype(jnp.int32).reshape(NW * NCH, CHUNK)
    out = _make_sc_gather()(y2, s2, table)
    return out.reshape(B, 2 * D)
